# baseline (device time: 35822 ns/iter reference)
import jax
import jax.numpy as jnp
from jax import lax
from jax.experimental import pallas as pl
from jax.experimental.pallas import tpu as pltpu

N_DEV = 4
N_LAYERS = 3


def kernel(x, Win0, Wout0, Win1, Wout1, Win2, Wout2):
    b, d_shard = x.shape
    _, hdim = Win0.shape
    _, d_out = Wout0.shape

    def body(x_hbm, win0_hbm, wout0_hbm, win1_hbm, wout1_hbm, win2_hbm,
             wout2_hbm, out_ref, xv, w0v, o0v, w1v, o1v, w2v, o2v,
             comm_ref, copy_sems, send_sems, recv_sems):
        my = lax.axis_index("i")
        p_y = jnp.bitwise_xor(my, 1)
        p_x = 3 - my
        p_d = jnp.bitwise_xor(my, 2)

        srcs = [x_hbm, win0_hbm, wout0_hbm, win1_hbm, wout1_hbm, win2_hbm,
                wout2_hbm]
        dsts = [xv, w0v, o0v, w1v, o1v, w2v, o2v]
        copies = []
        for i, (s, d) in enumerate(zip(srcs, dsts)):
            c = pltpu.make_async_copy(s, d, copy_sems.at[i])
            c.start()
            copies.append(c)

        barrier_sem = pltpu.get_barrier_semaphore()
        for p in [p_y, p_x, p_d]:
            pl.semaphore_signal(
                barrier_sem, inc=1,
                device_id=(p,), device_id_type=pl.DeviceIdType.MESH,
            )
        pl.semaphore_wait(barrier_sem, N_DEV - 1)

        win_vmem = [w0v, w1v, w2v]
        wout_vmem = [o0v, o1v, o2v]

        pending = []
        copies[0].wait()
        x_cur = xv[...].astype(jnp.bfloat16)
        for l in range(N_LAYERS):
            copies[1 + 2 * l].wait()
            partial = jnp.dot(
                x_cur, win_vmem[l][...].astype(jnp.bfloat16),
                preferred_element_type=jnp.float32,
            ).astype(jnp.bfloat16)
            base = 4 * l
            comm_ref[base] = partial
            rdmas = []
            for d, p in [(1, p_d), (2, p_y), (3, p_x)]:
                r = pltpu.make_async_remote_copy(
                    src_ref=comm_ref.at[base],
                    dst_ref=comm_ref.at[base + d],
                    send_sem=send_sems.at[3 * l + d - 1],
                    recv_sem=recv_sems.at[3 * l + d - 1],
                    device_id=(p,),
                    device_id_type=pl.DeviceIdType.MESH,
                )
                r.start()
                rdmas.append(r)
            for r in rdmas:
                r.wait_recv()
            pending.extend(rdmas)

            acc = (
                (partial + comm_ref[base + 1])
                + (comm_ref[base + 2] + comm_ref[base + 3])
            )
            h_full = jnp.maximum(acc, jnp.bfloat16(0.0))
            copies[2 + 2 * l].wait()
            if l < N_LAYERS - 1:
                x_cur = jnp.dot(
                    h_full, wout_vmem[l][...].astype(jnp.bfloat16),
                    preferred_element_type=jnp.float32,
                ).astype(jnp.bfloat16)
            else:
                out_ref[...] = jnp.dot(
                    h_full, wout_vmem[l][...].astype(jnp.bfloat16),
                    preferred_element_type=jnp.float32,
                )

        for r in pending:
            r.wait_send()

    out_shape = jax.ShapeDtypeStruct((b, d_out), jnp.float32)
    return pl.pallas_call(
        body,
        out_shape=out_shape,
        in_specs=[pl.BlockSpec(memory_space=pl.ANY)] * 7,
        out_specs=pl.BlockSpec(memory_space=pltpu.VMEM),
        scratch_shapes=[
            pltpu.VMEM((b, d_shard), jnp.float32),
            pltpu.VMEM((d_shard, hdim), jnp.float32),
            pltpu.VMEM((hdim, d_out), jnp.float32),
            pltpu.VMEM((d_shard, hdim), jnp.float32),
            pltpu.VMEM((hdim, d_out), jnp.float32),
            pltpu.VMEM((d_shard, hdim), jnp.float32),
            pltpu.VMEM((hdim, d_out), jnp.float32),
            pltpu.VMEM((4 * N_LAYERS, b, hdim), jnp.bfloat16),
            pltpu.SemaphoreType.DMA((7,)),
            pltpu.SemaphoreType.DMA((3 * N_LAYERS,)),
            pltpu.SemaphoreType.DMA((3 * N_LAYERS,)),
        ],
        compiler_params=pltpu.CompilerParams(collective_id=0),
    )(x, Win0, Wout0, Win1, Wout1, Win2, Wout2)


# device time: 35678 ns/iter; 1.0040x vs baseline; 1.0040x over previous
import jax
import jax.numpy as jnp
from jax import lax
from jax.experimental import pallas as pl
from jax.experimental.pallas import tpu as pltpu

N_DEV = 4
N_LAYERS = 3


def kernel(x, Win0, Wout0, Win1, Wout1, Win2, Wout2):
    b, d_shard = x.shape
    _, hdim = Win0.shape
    _, d_out = Wout0.shape

    def body(x_hbm, win0_hbm, wout0_hbm, win1_hbm, wout1_hbm, win2_hbm,
             wout2_hbm, out_ref, xv, w0v, o0v, w1v, o1v, w2v, o2v,
             comm_ref, copy_sems, send_sems, recv_sems):
        my = lax.axis_index("i")
        p_y = jnp.bitwise_xor(my, 1)
        p_x = 3 - my
        p_d = jnp.bitwise_xor(my, 2)

        srcs = [x_hbm, win0_hbm, wout0_hbm, win1_hbm, wout1_hbm, win2_hbm,
                wout2_hbm]
        dsts = [xv, w0v, o0v, w1v, o1v, w2v, o2v]
        copies = [
            pltpu.make_async_copy(s, d, copy_sems.at[i])
            for i, (s, d) in enumerate(zip(srcs, dsts))
        ]
        copies[0].start()
        copies[1].start()

        barrier_sem = pltpu.get_barrier_semaphore()
        for p in [p_y, p_x, p_d]:
            pl.semaphore_signal(
                barrier_sem, inc=1,
                device_id=(p,), device_id_type=pl.DeviceIdType.MESH,
            )
        pl.semaphore_wait(barrier_sem, N_DEV - 1)

        win_vmem = [w0v, w1v, w2v]
        wout_vmem = [o0v, o1v, o2v]

        pending = []
        copies[0].wait()
        copies[1].wait()
        x_cur = xv[...].astype(jnp.bfloat16)
        for l in range(N_LAYERS):
            if l > 0:
                copies[1 + 2 * l].wait()
            partial = jnp.dot(
                x_cur, win_vmem[l][...].astype(jnp.bfloat16),
                preferred_element_type=jnp.float32,
            ).astype(jnp.bfloat16)
            base = 4 * l
            comm_ref[base] = partial
            rdmas = []
            for d, p in [(1, p_d), (2, p_y), (3, p_x)]:
                r = pltpu.make_async_remote_copy(
                    src_ref=comm_ref.at[base],
                    dst_ref=comm_ref.at[base + d],
                    send_sem=send_sems.at[3 * l + d - 1],
                    recv_sem=recv_sems.at[3 * l + d - 1],
                    device_id=(p,),
                    device_id_type=pl.DeviceIdType.MESH,
                )
                r.start()
                rdmas.append(r)
            copies[2 + 2 * l].start()
            if l < N_LAYERS - 1:
                copies[3 + 2 * l].start()
            for r in rdmas:
                r.wait_recv()
            pending.extend(rdmas)

            acc = (
                (partial + comm_ref[base + 1])
                + (comm_ref[base + 2] + comm_ref[base + 3])
            )
            h_full = jnp.maximum(acc, jnp.bfloat16(0.0))
            copies[2 + 2 * l].wait()
            if l < N_LAYERS - 1:
                x_cur = jnp.dot(
                    h_full, wout_vmem[l][...].astype(jnp.bfloat16),
                    preferred_element_type=jnp.float32,
                ).astype(jnp.bfloat16)
            else:
                out_ref[...] = jnp.dot(
                    h_full, wout_vmem[l][...].astype(jnp.bfloat16),
                    preferred_element_type=jnp.float32,
                )

        for r in pending:
            r.wait_send()

    out_shape = jax.ShapeDtypeStruct((b, d_out), jnp.float32)
    return pl.pallas_call(
        body,
        out_shape=out_shape,
        in_specs=[pl.BlockSpec(memory_space=pl.ANY)] * 7,
        out_specs=pl.BlockSpec(memory_space=pltpu.VMEM),
        scratch_shapes=[
            pltpu.VMEM((b, d_shard), jnp.float32),
            pltpu.VMEM((d_shard, hdim), jnp.float32),
            pltpu.VMEM((hdim, d_out), jnp.float32),
            pltpu.VMEM((d_shard, hdim), jnp.float32),
            pltpu.VMEM((hdim, d_out), jnp.float32),
            pltpu.VMEM((d_shard, hdim), jnp.float32),
            pltpu.VMEM((hdim, d_out), jnp.float32),
            pltpu.VMEM((4 * N_LAYERS, b, hdim), jnp.bfloat16),
            pltpu.SemaphoreType.DMA((7,)),
            pltpu.SemaphoreType.DMA((3 * N_LAYERS,)),
            pltpu.SemaphoreType.DMA((3 * N_LAYERS,)),
        ],
        compiler_params=pltpu.CompilerParams(collective_id=0),
    )(x, Win0, Wout0, Win1, Wout1, Win2, Wout2)


# device time: 35059 ns/iter; 1.0218x vs baseline; 1.0177x over previous
import jax
import jax.numpy as jnp
from jax import lax
from jax.experimental import pallas as pl
from jax.experimental.pallas import tpu as pltpu

N_DEV = 4
N_LAYERS = 3


def kernel(x, Win0, Wout0, Win1, Wout1, Win2, Wout2):
    b, d_shard = x.shape
    _, hdim = Win0.shape
    _, d_out = Wout0.shape

    def body(x_ref, win0_ref, wout0_ref, win1_ref, wout1_ref, win2_ref,
             wout2_ref, out_ref, comm_ref, send_sems, recv_sems):
        my = lax.axis_index("i")
        p_y = jnp.bitwise_xor(my, 1)
        p_x = 3 - my
        p_d = jnp.bitwise_xor(my, 2)

        barrier_sem = pltpu.get_barrier_semaphore()
        for p in [p_y, p_x, p_d]:
            pl.semaphore_signal(
                barrier_sem, inc=1,
                device_id=(p,), device_id_type=pl.DeviceIdType.MESH,
            )
        pl.semaphore_wait(barrier_sem, N_DEV - 1)

        win_refs = [win0_ref, win1_ref, win2_ref]
        wout_refs = [wout0_ref, wout1_ref, wout2_ref]

        pending = []
        x_cur = x_ref[...].astype(jnp.bfloat16)
        for l in range(N_LAYERS):
            partial = jnp.dot(
                x_cur, win_refs[l][...].astype(jnp.bfloat16),
                preferred_element_type=jnp.float32,
            ).astype(jnp.bfloat16)
            base = 4 * l
            comm_ref[base] = partial
            rdmas = []
            for d, p in [(1, p_d), (2, p_y), (3, p_x)]:
                r = pltpu.make_async_remote_copy(
                    src_ref=comm_ref.at[base],
                    dst_ref=comm_ref.at[base + d],
                    send_sem=send_sems.at[3 * l + d - 1],
                    recv_sem=recv_sems.at[3 * l + d - 1],
                    device_id=(p,),
                    device_id_type=pl.DeviceIdType.MESH,
                )
                r.start()
                rdmas.append(r)
            for r in rdmas:
                r.wait_recv()
            pending.extend(rdmas)

            acc = (
                (partial + comm_ref[base + 1])
                + (comm_ref[base + 2] + comm_ref[base + 3])
            )
            h_full = jnp.maximum(acc, jnp.bfloat16(0.0))
            nxt = jnp.dot(
                h_full, wout_refs[l][...].astype(jnp.bfloat16),
                preferred_element_type=jnp.float32,
            ).astype(jnp.bfloat16)
            if l < N_LAYERS - 1:
                x_cur = nxt
            else:
                out_ref[...] = nxt

        for r in pending:
            r.wait_send()

    out_shape = jax.ShapeDtypeStruct((b, d_out), jnp.bfloat16)
    return pl.pallas_call(
        body,
        out_shape=out_shape,
        in_specs=[pl.BlockSpec(memory_space=pltpu.VMEM)] * 7,
        out_specs=pl.BlockSpec(memory_space=pltpu.VMEM),
        scratch_shapes=[
            pltpu.VMEM((4 * N_LAYERS, b, hdim), jnp.bfloat16),
            pltpu.SemaphoreType.DMA((3 * N_LAYERS,)),
            pltpu.SemaphoreType.DMA((3 * N_LAYERS,)),
        ],
        compiler_params=pltpu.CompilerParams(collective_id=0),
    )(x, Win0, Wout0, Win1, Wout1, Win2, Wout2)
